# baseline (speedup 1.0000x reference)
import jax
import jax.numpy as jnp
from jax import lax
from jax.experimental import pallas as pl
from jax.experimental.pallas import tpu as pltpu
from jax.experimental.pallas import tpu_sc as plsc

NUM_CLASSES = 1000000
DIM = 16
BATCH = 16384

_INFO = plsc.get_sparse_core_info()
_NC, _NS, _L = _INFO.num_cores, _INFO.num_subcores, _INFO.num_lanes
_NW = _NC * _NS
_BPW = BATCH // _NW
_CHUNK = 128
_NCHUNK = _BPW // _CHUNK
_NGRP = _BPW // _L


def _sc_gather(y_hbm, embf_hbm, outf_hbm, idx_m, eidx, rows_o, sem):
    wid = lax.axis_index("s") * _NC + lax.axis_index("c")
    base = wid * _BPW
    pltpu.sync_copy(y_hbm.at[pl.ds(base, _BPW)], idx_m)
    null_id = jnp.full((_L,), NUM_CLASSES, dtype=jnp.int32)
    iota = lax.iota(jnp.int32, _L)
    for i in range(_NGRP):
        v = idx_m[pl.ds(i * _L, _L)]
        y2 = jnp.where(v < 0, null_id, v) << 4
        for k in range(_L):
            eidx[pl.ds((i * _L + k) * _L, _L)] = iota + y2[k]
    copies = [
        pltpu.async_copy(
            embf_hbm.at[eidx.at[pl.ds(j * _CHUNK, _CHUNK)]],
            rows_o.at[pl.ds(j * _CHUNK, _CHUNK)],
            sem,
        )
        for j in range(_BPW * DIM // _CHUNK)
    ]
    for c in copies:
        c.wait()
    pltpu.sync_copy(rows_o, outf_hbm.at[pl.ds(base * DIM, _BPW * DIM)])


@jax.jit
def kernel(y, emb):
    mesh = plsc.VectorSubcoreMesh(core_axis_name="c", subcore_axis_name="s")
    run = pl.kernel(
        _sc_gather,
        mesh=mesh,
        out_type=jax.ShapeDtypeStruct((BATCH * DIM,), jnp.float32),
        scratch_types=[
            pltpu.VMEM((_BPW,), jnp.int32),
            pltpu.VMEM((_BPW * DIM,), jnp.int32),
            pltpu.VMEM((_BPW * DIM,), jnp.float32),
            pltpu.SemaphoreType.DMA,
        ],
        compiler_params=pltpu.CompilerParams(needs_layout_passes=False),
    )
    return run(y.astype(jnp.int32), emb.reshape(-1)).reshape(BATCH, DIM)


# speedup vs baseline: 1.6746x; 1.6746x over previous
import jax
import jax.numpy as jnp
from jax import lax
from jax.experimental import pallas as pl
from jax.experimental.pallas import tpu as pltpu
from jax.experimental.pallas import tpu_sc as plsc

NUM_CLASSES = 1000000
DIM = 16
BATCH = 16384

_INFO = plsc.get_sparse_core_info()
_NC, _NS, _L = _INFO.num_cores, _INFO.num_subcores, _INFO.num_lanes
_NW = _NC * _NS
_BPW = BATCH // _NW
_NGRP = _BPW // _L


def _sc_gather(y_hbm, emb_hbm, out_hbm, idx_m, rows_o, sem):
    wid = lax.axis_index("s") * _NC + lax.axis_index("c")
    base = wid * _BPW
    pltpu.sync_copy(y_hbm.at[pl.ds(base, _BPW)], idx_m)
    null_id = jnp.full((_L,), NUM_CLASSES, dtype=jnp.int32)
    copies = []
    for i in range(_NGRP):
        v = idx_m[pl.ds(i * _L, _L)]
        y2 = jnp.where(v < 0, null_id, v)
        for k in range(_L):
            r = i * _L + k
            copies.append(pltpu.async_copy(
                emb_hbm.at[pl.ds(y2[k], 1), :],
                rows_o.at[pl.ds(r, 1), :],
                sem,
            ))
    for c in copies:
        c.wait()
    pltpu.sync_copy(rows_o, out_hbm.at[pl.ds(base, _BPW)])


@jax.jit
def kernel(y, emb):
    mesh = plsc.VectorSubcoreMesh(core_axis_name="c", subcore_axis_name="s")
    run = pl.kernel(
        _sc_gather,
        mesh=mesh,
        out_type=jax.ShapeDtypeStruct((BATCH, DIM), jnp.float32),
        scratch_types=[
            pltpu.VMEM((_BPW,), jnp.int32),
            pltpu.VMEM((_BPW, DIM), jnp.float32),
            pltpu.SemaphoreType.DMA,
        ],
        compiler_params=pltpu.CompilerParams(needs_layout_passes=False),
    )
    return run(y.astype(jnp.int32), emb)


# speedup vs baseline: 6.4764x; 3.8674x over previous
import jax
import jax.numpy as jnp
from jax import lax
from jax.experimental import pallas as pl
from jax.experimental.pallas import tpu as pltpu
from jax.experimental.pallas import tpu_sc as plsc

NUM_CLASSES = 1000000
DIM = 16
BATCH = 16384

_INFO = plsc.get_sparse_core_info()
_NC, _NS, _L = _INFO.num_cores, _INFO.num_subcores, _INFO.num_lanes
_NW = _NC * _NS
_BPW = BATCH // _NW
_NGRP = _BPW // _L
_CW = 128


def _fire_wave(embt_hbm, cbuf, sem, y2, buf):
    cps = []
    for k in range(_L):
        s = y2[k]
        off = pl.multiple_of((s >> 7) * _CW, _CW)
        dst_row = (buf * _L + k) * DIM
        cps.append(pltpu.async_copy(
            embt_hbm.at[:, pl.ds(off, _CW)],
            cbuf.at[pl.ds(dst_row, DIM), :],
            sem,
        ))
    return cps


def _sc_gather(y_hbm, embt_hbm, outt_hbm, idx_m, cbuf, rows_t, sem):
    wid = lax.axis_index("s") * _NC + lax.axis_index("c")
    base = pl.multiple_of(wid * _BPW, _CW)
    pltpu.sync_copy(y_hbm.at[pl.ds(base, _BPW)], idx_m)
    null_id = jnp.full((_L,), NUM_CLASSES, dtype=jnp.int32)
    y2s = []
    for i in range(_NGRP):
        v = idx_m[pl.ds(i * _L, _L)]
        y2s.append(jnp.where(v < 0, null_id, v))
    iota = lax.iota(jnp.int32, _L)
    pending = _fire_wave(embt_hbm, cbuf, sem, y2s[0], 0)
    for w in range(_NGRP):
        nxt = None
        if w + 1 < _NGRP:
            nxt = _fire_wave(embt_hbm, cbuf, sem, y2s[w + 1], (w + 1) % 2)
        for c in pending:
            c.wait()
        buf = w % 2
        colv = y2s[w] & (_CW - 1)
        pos = iota + (w * _L)
        rowb = iota * DIM + (buf * _L * DIM)
        for c in range(DIM):
            val = plsc.load_gather(cbuf, [rowb + c, colv])
            plsc.store_scatter(rows_t, [jnp.full((_L,), c, jnp.int32), pos], val)
        pending = nxt
    pltpu.sync_copy(rows_t, outt_hbm.at[:, pl.ds(base, _BPW)])


@jax.jit
def kernel(y, emb):
    mesh = plsc.VectorSubcoreMesh(core_axis_name="c", subcore_axis_name="s")
    run = pl.kernel(
        _sc_gather,
        mesh=mesh,
        out_type=jax.ShapeDtypeStruct((DIM, BATCH), jnp.float32),
        scratch_types=[
            pltpu.VMEM((_BPW,), jnp.int32),
            pltpu.VMEM((2 * _L * DIM, _CW), jnp.float32),
            pltpu.VMEM((DIM, _BPW), jnp.float32),
            pltpu.SemaphoreType.DMA,
        ],
        compiler_params=pltpu.CompilerParams(needs_layout_passes=False),
    )
    return run(y.astype(jnp.int32), emb.T).T
